# Initial kernel scaffold; baseline (speedup 1.0000x reference)
import jax
import jax.numpy as jnp
from jax.experimental import pallas as pl


def kernel(x, edge_index, W, b):
    raise NotImplementedError("write your pallas kernel here")


# speedup vs baseline: 5.3016x; 5.3016x over previous
import functools

import jax
import jax.numpy as jnp
from jax import lax
from jax.experimental import pallas as pl
from jax.experimental.pallas import tpu as pltpu
from jax.experimental.pallas import tpu_sc as plsc

_N = 10000
_NP = 10240
_DIN = 128
_DH = 64
_NC = 2
_NS = 16
_B = 128
_ROWS_PER_SUB = _NP // _NS


def _project(x, W, b):
    n, d = x.shape
    blk = 80
    nblk = n // blk
    npblk = _NP // blk

    def body(x_ref, w_ref, b_ref, o_ref):
        o_ref[...] = jnp.dot(x_ref[...], w_ref[0],
                             preferred_element_type=jnp.float32) + b_ref[0]

    w_s = W.reshape(d, _NC, _DH).transpose(1, 0, 2)
    b_s = b.reshape(_NC, 1, _DH)
    return pl.pallas_call(
        body,
        grid=(nblk, _NC),
        in_specs=[
            pl.BlockSpec((blk, d), lambda i, c: (i, 0)),
            pl.BlockSpec((1, d, _DH), lambda i, c: (c, 0, 0)),
            pl.BlockSpec((1, 1, _DH), lambda i, c: (c, 0, 0)),
        ],
        out_specs=pl.BlockSpec((blk, _DH), lambda i, c: (c * npblk + i, 0)),
        out_shape=jax.ShapeDtypeStruct((_NC * _NP, _DH), jnp.float32),
    )(x, w_s, b_s)


def _propagate(hs, srcb, dst3, zeros):
    k = srcb.shape[-2]
    mesh = plsc.VectorSubcoreMesh(core_axis_name="c", subcore_axis_name="s",
                                  num_cores=_NC, num_subcores=_NS)
    out_ty = jax.ShapeDtypeStruct((_NC * _NP, _DH), jnp.float32)

    @functools.partial(
        pl.kernel,
        out_type=(out_ty, out_ty),
        mesh=mesh,
        compiler_params=pltpu.CompilerParams(use_tc_tiling_on_sc=False),
        scratch_types=[
            pltpu.VMEM((k, _B), jnp.int32),
            pltpu.VMEM((k, _B), jnp.int32),
            pltpu.VMEM((_B, _DH), jnp.float32),
            pltpu.VMEM_SHARED((_NP, _DH), jnp.float32),
        ],
    )
    def run(hs_ref, srcb_ref, dst3_ref, z_ref, out_ref, t_ref,
            src_v, dst_v, rows_v, acc):
        cid = lax.axis_index("c")
        sid = lax.axis_index("s")
        pltpu.sync_copy(srcb_ref.at[cid, sid], src_v)
        pltpu.sync_copy(dst3_ref.at[sid], dst_v)
        r0 = sid * _ROWS_PER_SUB

        def one_round(tab_in, tab_out):
            pltpu.sync_copy(z_ref.at[pl.ds(r0, _ROWS_PER_SUB)],
                            acc.at[pl.ds(r0, _ROWS_PER_SUB)])
            plsc.subcore_barrier()

            @pl.loop(0, k)
            def _(j):
                pltpu.sync_copy(tab_in.at[src_v.at[j]], rows_v)
                pltpu.sync_copy(rows_v, acc.at[dst_v.at[j]], add=True)

            plsc.subcore_barrier()
            pltpu.sync_copy(
                acc.at[pl.ds(r0, _ROWS_PER_SUB)],
                tab_out.at[pl.ds(cid * _NP + r0, _ROWS_PER_SUB)])
            plsc.subcore_barrier()

        one_round(hs_ref, out_ref)
        one_round(out_ref, t_ref)
        one_round(t_ref, out_ref)

    return run(hs, srcb, dst3, zeros)


def kernel(x, edge_index, W, b):
    hs = _project(x, W, b)
    src = edge_index[0]
    dst = edge_index[1]
    e = src.shape[0]
    k = -(-e // (_NS * _B))
    pad = _NS * _B * k - e
    src_p = jnp.concatenate([src, jnp.zeros((pad,), jnp.int32)])
    dst_p = jnp.concatenate([dst, jnp.full((pad,), _N, jnp.int32)])
    src3 = src_p.reshape(_NS, k, _B)
    dst3 = dst_p.reshape(_NS, k, _B)
    srcb = jnp.stack([src3, src3 + _NP])
    zeros = jnp.zeros((_NP, _DH), jnp.float32)
    out, _ = _propagate(hs, srcb, dst3, zeros)
    return jnp.concatenate([out[:_N], out[_NP:_NP + _N]], axis=1)


# speedup vs baseline: 6.1012x; 1.1508x over previous
import functools

import jax
import jax.numpy as jnp
from jax import lax
from jax.experimental import pallas as pl
from jax.experimental.pallas import tpu as pltpu
from jax.experimental.pallas import tpu_sc as plsc

_N = 10000
_NP = 10240
_DIN = 128
_DH = 64
_NC = 2
_NS = 16
_B = 128
_ROWS_PER_SUB = _NP // _NS


def _project(x, W, b):
    n, d = x.shape
    blk = 80
    nblk = n // blk
    npblk = _NP // blk

    def body(x_ref, w_ref, b_ref, o_ref):
        o_ref[...] = jnp.dot(x_ref[...], w_ref[0],
                             preferred_element_type=jnp.float32) + b_ref[0]

    w_s = W.reshape(d, _NC, _DH).transpose(1, 0, 2)
    b_s = b.reshape(_NC, 1, _DH)
    return pl.pallas_call(
        body,
        grid=(nblk, _NC),
        in_specs=[
            pl.BlockSpec((blk, d), lambda i, c: (i, 0)),
            pl.BlockSpec((1, d, _DH), lambda i, c: (c, 0, 0)),
            pl.BlockSpec((1, 1, _DH), lambda i, c: (c, 0, 0)),
        ],
        out_specs=pl.BlockSpec((blk, _DH), lambda i, c: (c * npblk + i, 0)),
        out_shape=jax.ShapeDtypeStruct((_NC * _NP, _DH), jnp.float32),
    )(x, w_s, b_s)


def _propagate(hs, srcb, dst3, zeros):
    k = srcb.shape[-2]
    mesh = plsc.VectorSubcoreMesh(core_axis_name="c", subcore_axis_name="s",
                                  num_cores=_NC, num_subcores=_NS)
    out_ty = jax.ShapeDtypeStruct((_NC * _NP, _DH), jnp.float32)

    @functools.partial(
        pl.kernel,
        out_type=(out_ty, out_ty),
        mesh=mesh,
        compiler_params=pltpu.CompilerParams(use_tc_tiling_on_sc=False),
        scratch_types=[
            pltpu.VMEM((k, _B), jnp.int32),
            pltpu.VMEM((k, _B), jnp.int32),
            pltpu.VMEM((_B, _DH), jnp.float32),
            pltpu.VMEM((_B, _DH), jnp.float32),
            pltpu.SemaphoreType.DMA,
            pltpu.SemaphoreType.DMA,
            pltpu.VMEM_SHARED((_NP, _DH), jnp.float32),
        ],
    )
    def run(hs_ref, srcb_ref, dst3_ref, z_ref, out_ref, t_ref,
            src_v, dst_v, rows0, rows1, sem0, sem1, acc):
        cid = lax.axis_index("c")
        sid = lax.axis_index("s")
        pltpu.sync_copy(srcb_ref.at[cid, sid], src_v)
        pltpu.sync_copy(dst3_ref.at[sid], dst_v)
        r0 = sid * _ROWS_PER_SUB

        def one_round(tab_in, tab_out):
            pltpu.sync_copy(z_ref.at[pl.ds(r0, _ROWS_PER_SUB)],
                            acc.at[pl.ds(r0, _ROWS_PER_SUB)])
            plsc.subcore_barrier()

            pltpu.async_copy(tab_in.at[src_v.at[0]], rows0, sem0)

            @pl.loop(0, k // 2)
            def _(i):
                j = 2 * i
                pltpu.async_copy(tab_in.at[src_v.at[j + 1]], rows1, sem1)
                pltpu.make_async_copy(tab_in.at[src_v.at[j]], rows0,
                                      sem0).wait()
                pltpu.sync_copy(rows0, acc.at[dst_v.at[j]], add=True)

                @pl.when(j + 2 < k)
                def _():
                    pltpu.async_copy(tab_in.at[src_v.at[j + 2]], rows0, sem0)

                pltpu.make_async_copy(tab_in.at[src_v.at[j + 1]], rows1,
                                      sem1).wait()
                pltpu.sync_copy(rows1, acc.at[dst_v.at[j + 1]], add=True)

            plsc.subcore_barrier()
            pltpu.sync_copy(
                acc.at[pl.ds(r0, _ROWS_PER_SUB)],
                tab_out.at[pl.ds(cid * _NP + r0, _ROWS_PER_SUB)])
            plsc.subcore_barrier()

        one_round(hs_ref, out_ref)
        one_round(out_ref, t_ref)
        one_round(t_ref, out_ref)

    return run(hs, srcb, dst3, zeros)


def kernel(x, edge_index, W, b):
    hs = _project(x, W, b)
    src = edge_index[0]
    dst = edge_index[1]
    e = src.shape[0]
    k = -(-e // (_NS * _B))
    k += k % 2
    pad = _NS * _B * k - e
    src_p = jnp.concatenate([src, jnp.zeros((pad,), jnp.int32)])
    dst_p = jnp.concatenate([dst, jnp.full((pad,), _N, jnp.int32)])
    src3 = src_p.reshape(_NS, k, _B)
    dst3 = dst_p.reshape(_NS, k, _B)
    srcb = jnp.stack([src3, src3 + _NP])
    zeros = jnp.zeros((_NP, _DH), jnp.float32)
    out, _ = _propagate(hs, srcb, dst3, zeros)
    return jnp.concatenate([out[:_N], out[_NP:_NP + _N]], axis=1)
